# Initial kernel scaffold; baseline (speedup 1.0000x reference)
import jax
import jax.numpy as jnp
from jax.experimental import pallas as pl


def kernel(x, edge_index, W_emb, b_emb, Wl0, bl0, Wr0, Wl1, bl1, Wr1, Wl2, bl2, Wr2, Wl3, bl3, Wr3):
    raise NotImplementedError("write your pallas kernel here")


# speedup vs baseline: 4.1463x; 4.1463x over previous
import functools

import jax
import jax.numpy as jnp
from jax import lax
from jax.experimental import pallas as pl
from jax.experimental.pallas import tpu as pltpu
from jax.experimental.pallas import tpu_sc as plsc

N = 10000
NP = 10240
E = 320000
H = 128
NC = 2
NS = 16
NW = NC * NS
EPW = E // NW
C = 80
NCHUNK = EPW // C
RPT = NP // NS

_HIGH = lax.Precision.HIGHEST

_mesh = plsc.VectorSubcoreMesh(
    core_axis_name="c", subcore_axis_name="s", num_cores=NC, num_subcores=NS)


@functools.partial(
    pl.kernel,
    mesh=_mesh,
    out_type=jax.ShapeDtypeStruct((2 * NP, H), jnp.float32),
    scratch_types=[
        pltpu.VMEM((C,), jnp.int32),
        pltpu.VMEM((C,), jnp.int32),
        pltpu.VMEM((C, H), jnp.float32),
        pltpu.VMEM_SHARED((NP, H), jnp.float32),
        pltpu.SemaphoreType.DMA,
    ],
)
def _segsum_sc(p_hbm, src_hbm, dst_hbm, zeros_hbm, out_hbm,
               sidx, didx, rows, acc, sem):
    cid = lax.axis_index("c")
    sid = lax.axis_index("s")
    wid = cid * NS + sid
    pltpu.sync_copy(zeros_hbm.at[pl.ds(sid * RPT, RPT), :],
                    acc.at[pl.ds(sid * RPT, RPT), :])
    plsc.subcore_barrier()
    base = wid * EPW

    def body(i, carry):
        off = base + i * C
        pltpu.sync_copy(src_hbm.at[pl.ds(off, C)], sidx)
        pltpu.sync_copy(dst_hbm.at[pl.ds(off, C)], didx)
        pltpu.async_copy(p_hbm.at[sidx], rows, sem).wait()
        pltpu.sync_copy(rows, acc.at[didx], add=True)
        return carry

    lax.fori_loop(0, NCHUNK, body, 0)
    plsc.subcore_barrier()
    pltpu.sync_copy(acc.at[pl.ds(sid * RPT, RPT), :],
                    out_hbm.at[pl.ds(cid * NP + sid * RPT, RPT), :])


BN = 2000


def _emb_body(x_ref, w_ref, b_ref, h_ref):
    h_ref[...] = jax.nn.relu(
        jnp.dot(x_ref[...], w_ref[...], precision=_HIGH,
                preferred_element_type=jnp.float32) + b_ref[...])


def _pq_body(h_ref, wl_ref, wr_ref, p_ref, q_ref):
    h = h_ref[...]
    p_ref[...] = jnp.dot(h, wl_ref[...], precision=_HIGH,
                         preferred_element_type=jnp.float32)
    q_ref[...] = jnp.dot(h, wr_ref[...], precision=_HIGH,
                         preferred_element_type=jnp.float32)


def _deg_body(d0_ref, d1_ref, out_ref):
    out_ref[...] = jnp.maximum(d0_ref[...] + d1_ref[...], 1.0)


def _comb_pq_body(a0_ref, a1_ref, dg_ref, q_ref, b_ref, wl_ref, wr_ref,
                  h_ref, p_ref, qn_ref):
    hh = jax.nn.relu((a0_ref[...] + a1_ref[...]) / dg_ref[...]
                     + b_ref[...] + q_ref[...])
    h_ref[...] = hh
    p_ref[...] = jnp.dot(hh, wl_ref[...], precision=_HIGH,
                         preferred_element_type=jnp.float32)
    qn_ref[...] = jnp.dot(hh, wr_ref[...], precision=_HIGH,
                          preferred_element_type=jnp.float32)


def _comb_body(a0_ref, a1_ref, dg_ref, q_ref, b_ref, h_ref):
    h_ref[...] = jax.nn.relu((a0_ref[...] + a1_ref[...]) / dg_ref[...]
                             + b_ref[...] + q_ref[...])


def _row_spec():
    return pl.BlockSpec((BN, H), lambda i: (i, 0))


def _w_spec():
    return pl.BlockSpec((H, H), lambda i: (0, 0))


def _b_spec():
    return pl.BlockSpec((1, H), lambda i: (0, 0))


def _f32(shape):
    return jax.ShapeDtypeStruct(shape, jnp.float32)


def kernel(x, edge_index, W_emb, b_emb, Wl0, bl0, Wr0, Wl1, bl1, Wr1,
           Wl2, bl2, Wr2, Wl3, bl3, Wr3):
    grid = (N // BN,)
    src = edge_index[0]
    dst = edge_index[1]
    zeros = jnp.zeros((NP, H), jnp.float32)
    ones = jnp.ones((N, H), jnp.float32)

    h = pl.pallas_call(
        _emb_body, grid=grid,
        in_specs=[_row_spec(), _w_spec(), _b_spec()],
        out_specs=_row_spec(), out_shape=_f32((N, H)),
    )(x, W_emb, b_emb.reshape(1, H))

    dacc = _segsum_sc(ones, src, dst, zeros)
    degc = pl.pallas_call(
        _deg_body, grid=grid,
        in_specs=[_row_spec(), _row_spec()],
        out_specs=_row_spec(), out_shape=_f32((N, H)),
    )(dacc[:N], dacc[NP:NP + N])

    p, q = pl.pallas_call(
        _pq_body, grid=grid,
        in_specs=[_row_spec(), _w_spec(), _w_spec()],
        out_specs=[_row_spec(), _row_spec()],
        out_shape=[_f32((N, H)), _f32((N, H))],
    )(h, Wl0, Wr0)

    layer_b = [bl0, bl1, bl2, bl3]
    next_w = [(Wl1, Wr1), (Wl2, Wr2), (Wl3, Wr3), None]
    for li in range(4):
        acc = _segsum_sc(p, src, dst, zeros)
        bl = layer_b[li].reshape(1, H)
        a0, a1 = acc[:N], acc[NP:NP + N]
        if next_w[li] is not None:
            wl_n, wr_n = next_w[li]
            h, p, q = pl.pallas_call(
                _comb_pq_body, grid=grid,
                in_specs=[_row_spec(), _row_spec(), _row_spec(), _row_spec(),
                          _b_spec(), _w_spec(), _w_spec()],
                out_specs=[_row_spec(), _row_spec(), _row_spec()],
                out_shape=[_f32((N, H))] * 3,
            )(a0, a1, degc, q, bl, wl_n, wr_n)
        else:
            h = pl.pallas_call(
                _comb_body, grid=grid,
                in_specs=[_row_spec(), _row_spec(), _row_spec(), _row_spec(),
                          _b_spec()],
                out_specs=_row_spec(), out_shape=_f32((N, H)),
            )(a0, a1, degc, q, bl)
    return h
